# baseline (speedup 1.0000x reference)
import functools

import jax
import jax.numpy as jnp
from jax import lax
from jax.experimental import pallas as pl
from jax.experimental.pallas import tpu as pltpu
from jax.experimental.pallas import tpu_sc as plsc

N = 100000
E = 3200000
NC = 2
NS = 16
NW = NC * NS
CHUNK = 128
ROWS = E // CHUNK
KB = 8
ROWS_PER_W = (ROWS // (NW * KB)) * KB
EXTRA_TILES = (ROWS - ROWS_PER_W * NW) // KB
NPAD = 100352
SLICE = NPAD // NS
BUF = SLICE // 8

_mesh = plsc.VectorSubcoreMesh(core_axis_name="c", subcore_axis_name="s")


def _worker_blocks(c, s):
    wid = c * NS + s
    start = wid * ROWS_PER_W + KB * jnp.minimum(wid, EXTRA_TILES)
    nblk = ROWS_PER_W // KB + jnp.where(wid < EXTRA_TILES, 1, 0)
    return start, nblk


def _deg_body(dst_hbm, zeros_hbm, out_hbm, didx, ones_v, vbuf, deg_tbl,
              isem, ssem):
    c = lax.axis_index("c")
    s = lax.axis_index("s")
    o = s * SLICE
    for j in range(SLICE // BUF):
        pltpu.sync_copy(zeros_hbm.at[pl.ds(o + j * BUF, BUF)], vbuf)
        pltpu.sync_copy(vbuf, deg_tbl.at[pl.ds(o + j * BUF, BUF)])
    for i in range(CHUNK // 16):
        ones_v[pl.ds(i * 16, 16)] = jnp.ones((16,), jnp.float32)
    plsc.subcore_barrier()
    start, nblk = _worker_blocks(c, s)

    pltpu.async_copy(dst_hbm.at[pl.ds(start, KB)], didx.at[0], isem)
    pltpu.async_copy(dst_hbm.at[pl.ds(start + KB, KB)], didx.at[1], isem)

    @pl.loop(0, nblk)
    def _(b):
        q = b % 4
        pltpu.make_async_copy(dst_hbm.at[pl.ds(start, KB)], didx.at[q],
                              isem).wait()

        @pl.when(b >= 2)
        def _():
            for j in range(KB):
                pltpu.make_async_copy(zeros_hbm.at[pl.ds(0, CHUNK)],
                                      ones_v, ssem).wait()

        @pl.when(b + 2 < nblk)
        def _():
            pltpu.async_copy(dst_hbm.at[pl.ds(start + (b + 2) * KB, KB)],
                             didx.at[(b + 2) % 4], isem)

        for j in range(KB):
            pltpu.async_copy(ones_v, deg_tbl.at[didx.at[q, j]], ssem,
                             add=True)

    for j in range(2 * KB):
        pltpu.make_async_copy(zeros_hbm.at[pl.ds(0, CHUNK)], ones_v,
                              ssem).wait()

    plsc.subcore_barrier()
    for j in range(SLICE // BUF):
        pltpu.sync_copy(deg_tbl.at[pl.ds(o + j * BUF, BUF)], vbuf)
        pltpu.sync_copy(vbuf, out_hbm.at[pl.ds(c * NPAD + o + j * BUF, BUF)])


_deg_call = pl.kernel(
    _deg_body,
    out_type=jax.ShapeDtypeStruct((NC * NPAD,), jnp.float32),
    mesh=_mesh,
    scratch_types=[
        pltpu.VMEM((4, KB, CHUNK), jnp.int32),
        pltpu.VMEM((CHUNK,), jnp.float32),
        pltpu.VMEM((BUF,), jnp.float32),
        pltpu.VMEM_SHARED((NPAD,), jnp.float32),
        pltpu.SemaphoreType.DMA,
        pltpu.SemaphoreType.DMA,
    ],
)


def _msg_body(F, y_hbm, src_hbm, dst_hbm, zeros_hbm, out_hbm,
              sidx, didx, rows_v, vbuf, acc_tbl, isem, gsem, ssem):
    c = lax.axis_index("c")
    s = lax.axis_index("s")
    o = s * SLICE
    for j in range(SLICE // BUF):
        pltpu.sync_copy(zeros_hbm.at[pl.ds(o + j * BUF, BUF)], vbuf)
        pltpu.sync_copy(vbuf, acc_tbl.at[pl.ds(o + j * BUF, BUF)])
    plsc.subcore_barrier()
    start, nblk = _worker_blocks(c, s)

    for bb in range(2):
        pltpu.async_copy(src_hbm.at[pl.ds(start + bb * KB, KB)],
                         sidx.at[bb], isem)
        pltpu.async_copy(dst_hbm.at[pl.ds(start + bb * KB, KB)],
                         didx.at[bb], isem)

    @pl.loop(0, nblk)
    def _(b):
        q = b % 4
        p = b % 2
        pltpu.make_async_copy(src_hbm.at[pl.ds(start, KB)], sidx.at[q],
                              isem).wait()
        pltpu.make_async_copy(dst_hbm.at[pl.ds(start, KB)], didx.at[q],
                              isem).wait()

        @pl.when(b >= 2)
        def _():
            for j in range(KB):
                pltpu.make_async_copy(y_hbm.at[pl.ds(0, CHUNK)],
                                      rows_v.at[0, j], ssem).wait()

        @pl.when(b + 2 < nblk)
        def _():
            r2 = start + (b + 2) * KB
            pltpu.async_copy(src_hbm.at[pl.ds(r2, KB)],
                             sidx.at[(b + 2) % 4], isem)
            pltpu.async_copy(dst_hbm.at[pl.ds(r2, KB)],
                             didx.at[(b + 2) % 4], isem)

        gcps = [pltpu.async_copy(y_hbm.at[sidx.at[q, j]],
                                 rows_v.at[p, j], gsem)
                for j in range(KB)]
        for cp in gcps:
            cp.wait()

        for j in range(KB):
            pltpu.async_copy(rows_v.at[p, j], acc_tbl.at[didx.at[q, j]],
                             ssem, add=True)

    for j in range(2 * KB):
        pltpu.make_async_copy(y_hbm.at[pl.ds(0, CHUNK)], rows_v.at[0, 0],
                              ssem).wait()

    plsc.subcore_barrier()
    for j in range(SLICE // BUF):
        pltpu.sync_copy(acc_tbl.at[pl.ds(o + j * BUF, BUF)], vbuf)
        pltpu.sync_copy(vbuf, out_hbm.at[pl.ds(c * NPAD + o + j * BUF, BUF)])


def _make_msg_call(F):
    return pl.kernel(
        functools.partial(_msg_body, F),
        out_type=jax.ShapeDtypeStruct((NC * NPAD, F), jnp.float32),
        mesh=_mesh,
        compiler_params=pltpu.CompilerParams(use_tc_tiling_on_sc=False),
        scratch_types=[
            pltpu.VMEM((4, KB, CHUNK), jnp.int32),
            pltpu.VMEM((4, KB, CHUNK), jnp.int32),
            pltpu.VMEM((2, KB, CHUNK, F), jnp.float32),
            pltpu.VMEM((BUF, F), jnp.float32),
            pltpu.VMEM_SHARED((NPAD, F), jnp.float32),
            pltpu.SemaphoreType.DMA,
            pltpu.SemaphoreType.DMA,
            pltpu.SemaphoreType.DMA,
        ],
    )


_msg_call_8 = _make_msg_call(8)


NB = 3136
GRID = NPAD // NB


def _dis_T(degpT_blk):
    degT = degpT_blk.T
    return lax.rsqrt(degT[0:1, :] + degT[1:2, :] + 1.0)


def _dense1_body(x_ref, degpT_ref, W1T_ref, y1_ref):
    dis = _dis_T(degpT_ref[...])
    xT = x_ref[...].T
    w0 = W1T_ref[:, 0:1]
    w1 = W1T_ref[:, 1:2]
    y1T = (w0 * xT[0:1, :] + w1 * xT[1:2, :]) * dis
    y1_ref[...] = y1T.T


_dense1 = pl.pallas_call(
    _dense1_body,
    grid=(GRID,),
    in_specs=[
        pl.BlockSpec((NB, 2), lambda i: (i, 0)),
        pl.BlockSpec((NB, 2), lambda i: (i, 0)),
        pl.BlockSpec((8, 2), lambda i: (0, 0)),
    ],
    out_specs=pl.BlockSpec((NB, 8), lambda i: (i, 0)),
    out_shape=jax.ShapeDtypeStruct((NPAD, 8), jnp.float32),
)


def _dense2_body(a0_ref, a1_ref, y1_ref, degpT_ref, b1_ref, W2T_ref,
                 y2_ref):
    dis = _dis_T(degpT_ref[...])
    aT = a0_ref[...].T + a1_ref[...].T + y1_ref[...].T
    hT = jnp.maximum(aT * dis + b1_ref[...], 0.0)
    acc = jnp.zeros((2, NB), jnp.float32)
    for f in range(8):
        acc = acc + W2T_ref[:, f:f + 1] * hT[f:f + 1, :]
    y2T = jnp.concatenate([acc * dis, jnp.zeros((6, NB), jnp.float32)],
                          axis=0)
    y2_ref[...] = y2T.T


_dense2 = pl.pallas_call(
    _dense2_body,
    grid=(GRID,),
    in_specs=[
        pl.BlockSpec((NB, 8), lambda i: (i, 0)),
        pl.BlockSpec((NB, 8), lambda i: (i + GRID, 0)),
        pl.BlockSpec((NB, 8), lambda i: (i, 0)),
        pl.BlockSpec((NB, 2), lambda i: (i, 0)),
        pl.BlockSpec((8, 1), lambda i: (0, 0)),
        pl.BlockSpec((2, 8), lambda i: (0, 0)),
    ],
    out_specs=pl.BlockSpec((NB, 8), lambda i: (i, 0)),
    out_shape=jax.ShapeDtypeStruct((NPAD, 8), jnp.float32),
)


def _dense3_body(a0_ref, a1_ref, y2_ref, degpT_ref, b2_ref, out_ref):
    dis = _dis_T(degpT_ref[...])
    aT = a0_ref[...].T + a1_ref[...].T + y2_ref[...].T
    outT = aT[0:2, :] * dis + b2_ref[...]
    out_ref[...] = outT.T


_dense3 = pl.pallas_call(
    _dense3_body,
    grid=(GRID,),
    in_specs=[
        pl.BlockSpec((NB, 8), lambda i: (i, 0)),
        pl.BlockSpec((NB, 8), lambda i: (i + GRID, 0)),
        pl.BlockSpec((NB, 8), lambda i: (i, 0)),
        pl.BlockSpec((NB, 2), lambda i: (i, 0)),
        pl.BlockSpec((2, 1), lambda i: (0, 0)),
    ],
    out_specs=pl.BlockSpec((NB, 2), lambda i: (i, 0)),
    out_shape=jax.ShapeDtypeStruct((NPAD, 2), jnp.float32),
)


def kernel(x, edge_index, W1, b1, W2, b2):
    src = edge_index[0].astype(jnp.int32).reshape(ROWS, CHUNK)
    dst = edge_index[1].astype(jnp.int32).reshape(ROWS, CHUNK)
    zeros1 = jnp.zeros((NPAD,), jnp.float32)
    zeros8 = jnp.zeros((NPAD, 8), jnp.float32)

    degpT = _deg_call(dst, zeros1).reshape(NC, NPAD).T
    xp = jnp.pad(x, ((0, NPAD - N), (0, 0)))

    y1 = _dense1(xp, degpT, W1.T)

    acc1 = _msg_call_8(y1, src, dst, zeros8)
    y2 = _dense2(acc1, acc1, y1, degpT,
                 b1.reshape(8, 1), W2.T)

    acc2 = _msg_call_8(y2, src, dst, zeros8)
    outp = _dense3(acc2, acc2, y2, degpT, b2.reshape(2, 1))
    return outp[:N]


# speedup vs baseline: 1.6681x; 1.6681x over previous
import functools

import jax
import jax.numpy as jnp
from jax import lax
from jax.experimental import pallas as pl
from jax.experimental.pallas import tpu as pltpu
from jax.experimental.pallas import tpu_sc as plsc

N = 100000
E = 3200000
NC = 2
NS = 16
NW = NC * NS
CHUNK = 128
ROWS = E // CHUNK
KB = 8
ROWS_PER_W = (ROWS // (NW * KB)) * KB
EXTRA_TILES = (ROWS - ROWS_PER_W * NW) // KB
NPAD = 100352
SLICE = NPAD // NS
BUF = SLICE // 8

_mesh = plsc.VectorSubcoreMesh(core_axis_name="c", subcore_axis_name="s")


def _worker_blocks(c, s):
    wid = c * NS + s
    start = wid * ROWS_PER_W + KB * jnp.minimum(wid, EXTRA_TILES)
    nblk = ROWS_PER_W // KB + jnp.where(wid < EXTRA_TILES, 1, 0)
    return start, nblk


def _deg_body(dst_hbm, zeros_hbm, out_hbm, didx, ones_v, vbuf, deg_tbl,
              isem, ssem):
    c = lax.axis_index("c")
    s = lax.axis_index("s")
    o = s * SLICE
    for j in range(SLICE // BUF):
        pltpu.sync_copy(zeros_hbm.at[pl.ds(o + j * BUF, BUF)], vbuf)
        pltpu.sync_copy(vbuf, deg_tbl.at[pl.ds(o + j * BUF, BUF)])
    for i in range(CHUNK // 16):
        ones_v[pl.ds(i * 16, 16)] = jnp.ones((16,), jnp.float32)
    plsc.subcore_barrier()
    start, nblk = _worker_blocks(c, s)

    pltpu.async_copy(dst_hbm.at[pl.ds(start, KB)], didx.at[0], isem)
    pltpu.async_copy(dst_hbm.at[pl.ds(start + KB, KB)], didx.at[1], isem)

    @pl.loop(0, nblk)
    def _(b):
        q = b % 4
        pltpu.make_async_copy(dst_hbm.at[pl.ds(start, KB)], didx.at[q],
                              isem).wait()

        @pl.when(b >= 2)
        def _():
            for j in range(KB):
                pltpu.make_async_copy(zeros_hbm.at[pl.ds(0, CHUNK)],
                                      ones_v, ssem).wait()

        @pl.when(b + 2 < nblk)
        def _():
            pltpu.async_copy(dst_hbm.at[pl.ds(start + (b + 2) * KB, KB)],
                             didx.at[(b + 2) % 4], isem)

        for j in range(KB):
            pltpu.async_copy(ones_v, deg_tbl.at[didx.at[q, j]], ssem,
                             add=True)

    for j in range(2 * KB):
        pltpu.make_async_copy(zeros_hbm.at[pl.ds(0, CHUNK)], ones_v,
                              ssem).wait()

    plsc.subcore_barrier()
    for j in range(SLICE // BUF):
        pltpu.sync_copy(deg_tbl.at[pl.ds(o + j * BUF, BUF)], vbuf)
        pltpu.sync_copy(vbuf, out_hbm.at[pl.ds(c * NPAD + o + j * BUF, BUF)])


_deg_call = pl.kernel(
    _deg_body,
    out_type=jax.ShapeDtypeStruct((NC * NPAD,), jnp.float32),
    mesh=_mesh,
    scratch_types=[
        pltpu.VMEM((4, KB, CHUNK), jnp.int32),
        pltpu.VMEM((CHUNK,), jnp.float32),
        pltpu.VMEM((BUF,), jnp.float32),
        pltpu.VMEM_SHARED((NPAD,), jnp.float32),
        pltpu.SemaphoreType.DMA,
        pltpu.SemaphoreType.DMA,
    ],
)


def _msg_body(F, y_hbm, src_hbm, dst_hbm, zeros_hbm, out_hbm,
              sidx, didx, rows_v, vbuf, acc_tbl, y_tbl, isem, gsem, ssem):
    c = lax.axis_index("c")
    s = lax.axis_index("s")
    o = s * SLICE
    @pl.when(s < NS - 1)
    def _():
        for j in range(SLICE // BUF):
            pltpu.sync_copy(y_hbm.at[pl.ds(o + j * BUF, BUF)], vbuf)
            pltpu.sync_copy(vbuf, y_tbl.at[pl.ds(o + j * BUF, BUF)])

    @pl.when(s == NS - 1)
    def _():
        o15 = (NS - 1) * SLICE
        for j in range((N - o15) // BUF):
            pltpu.sync_copy(y_hbm.at[pl.ds(o15 + j * BUF, BUF)], vbuf)
            pltpu.sync_copy(vbuf, y_tbl.at[pl.ds(o15 + j * BUF, BUF)])
        rem = (N - o15) % BUF
        rbase = o15 + ((N - o15) // BUF) * BUF
        pltpu.sync_copy(y_hbm.at[pl.ds(rbase, rem)],
                        vbuf.at[pl.ds(0, rem)])
        pltpu.sync_copy(vbuf.at[pl.ds(0, rem)],
                        y_tbl.at[pl.ds(rbase, rem)])

    for j in range(SLICE // BUF):
        pltpu.sync_copy(zeros_hbm.at[pl.ds(o + j * BUF, BUF)], vbuf)
        pltpu.sync_copy(vbuf, acc_tbl.at[pl.ds(o + j * BUF, BUF)])
    plsc.subcore_barrier()
    start, nblk = _worker_blocks(c, s)

    for bb in range(2):
        pltpu.async_copy(src_hbm.at[pl.ds(start + bb * KB, KB)],
                         sidx.at[bb], isem)
        pltpu.async_copy(dst_hbm.at[pl.ds(start + bb * KB, KB)],
                         didx.at[bb], isem)

    @pl.loop(0, nblk)
    def _(b):
        q = b % 4
        p = b % 2
        pltpu.make_async_copy(src_hbm.at[pl.ds(start, KB)], sidx.at[q],
                              isem).wait()
        pltpu.make_async_copy(dst_hbm.at[pl.ds(start, KB)], didx.at[q],
                              isem).wait()

        @pl.when(b >= 2)
        def _():
            for j in range(KB):
                pltpu.make_async_copy(y_hbm.at[pl.ds(0, CHUNK)],
                                      rows_v.at[0, j], ssem).wait()

        @pl.when(b + 2 < nblk)
        def _():
            r2 = start + (b + 2) * KB
            pltpu.async_copy(src_hbm.at[pl.ds(r2, KB)],
                             sidx.at[(b + 2) % 4], isem)
            pltpu.async_copy(dst_hbm.at[pl.ds(r2, KB)],
                             didx.at[(b + 2) % 4], isem)

        gcps = [pltpu.async_copy(y_tbl.at[sidx.at[q, j]],
                                 rows_v.at[p, j], gsem)
                for j in range(KB)]
        for cp in gcps:
            cp.wait()

        for j in range(KB):
            pltpu.async_copy(rows_v.at[p, j], acc_tbl.at[didx.at[q, j]],
                             ssem, add=True)

    for j in range(2 * KB):
        pltpu.make_async_copy(y_hbm.at[pl.ds(0, CHUNK)], rows_v.at[0, 0],
                              ssem).wait()

    plsc.subcore_barrier()
    for j in range(SLICE // BUF):
        pltpu.sync_copy(acc_tbl.at[pl.ds(o + j * BUF, BUF)], vbuf)
        pltpu.sync_copy(vbuf, out_hbm.at[pl.ds(c * NPAD + o + j * BUF, BUF)])


def _make_msg_call(F):
    return pl.kernel(
        functools.partial(_msg_body, F),
        out_type=jax.ShapeDtypeStruct((NC * NPAD, F), jnp.float32),
        mesh=_mesh,
        compiler_params=pltpu.CompilerParams(use_tc_tiling_on_sc=False),
        scratch_types=[
            pltpu.VMEM((4, KB, CHUNK), jnp.int32),
            pltpu.VMEM((4, KB, CHUNK), jnp.int32),
            pltpu.VMEM((2, KB, CHUNK, F), jnp.float32),
            pltpu.VMEM((BUF, F), jnp.float32),
            pltpu.VMEM_SHARED((NPAD, F), jnp.float32),
            pltpu.VMEM_SHARED((N, F), jnp.float32),
            pltpu.SemaphoreType.DMA,
            pltpu.SemaphoreType.DMA,
            pltpu.SemaphoreType.DMA,
        ],
    )


_msg_call_8 = _make_msg_call(8)


def _dis(degp_ref):
    deg = degp_ref[0:1, :] + degp_ref[1:2, :] + 1.0
    return lax.rsqrt(deg)


def _dense1_body(xT_ref, degp_ref, W1T_ref, y1T_ref):
    dis = _dis(degp_ref)
    x0 = xT_ref[0:1, :]
    x1 = xT_ref[1:2, :]
    w0 = W1T_ref[:, 0:1]
    w1 = W1T_ref[:, 1:2]
    y1T_ref[...] = (w0 * x0 + w1 * x1) * dis


_dense1 = pl.pallas_call(
    _dense1_body,
    out_shape=jax.ShapeDtypeStruct((8, N), jnp.float32),
)


def _dense2_body(a0T_ref, a1T_ref, y1T_ref, degp_ref, b1_ref, W2T_ref,
                 y2T_ref):
    dis = _dis(degp_ref)
    hT = (a0T_ref[...] + a1T_ref[...] + y1T_ref[...]) * dis + b1_ref[...]
    hT = jnp.maximum(hT, 0.0)
    acc = jnp.zeros((2, N), jnp.float32)
    for f in range(8):
        acc = acc + W2T_ref[:, f:f + 1] * hT[f:f + 1, :]
    y2T_ref[0:2, :] = acc * dis
    y2T_ref[2:8, :] = jnp.zeros((6, N), jnp.float32)


_dense2 = pl.pallas_call(
    _dense2_body,
    out_shape=jax.ShapeDtypeStruct((8, N), jnp.float32),
)


def _dense3_body(a0T_ref, a1T_ref, y2T_ref, degp_ref, b2_ref, outT_ref):
    dis = _dis(degp_ref)
    outT_ref[...] = (a0T_ref[...] + a1T_ref[...] + y2T_ref[...]) * dis \
        + b2_ref[...]


_dense3 = pl.pallas_call(
    _dense3_body,
    out_shape=jax.ShapeDtypeStruct((2, N), jnp.float32),
)


def kernel(x, edge_index, W1, b1, W2, b2):
    src = edge_index[0].astype(jnp.int32).reshape(ROWS, CHUNK)
    dst = edge_index[1].astype(jnp.int32).reshape(ROWS, CHUNK)
    zeros1 = jnp.zeros((NPAD,), jnp.float32)
    zeros8 = jnp.zeros((NPAD, 8), jnp.float32)

    degp = _deg_call(dst, zeros1).reshape(NC, NPAD)[:, :N]

    xT = x.T
    y1T = _dense1(xT, degp, W1.T)
    y1 = y1T.T

    acc1 = _msg_call_8(y1, src, dst, zeros8).reshape(NC, NPAD, 8)[:, :N, :]
    y2T8 = _dense2(acc1[0].T, acc1[1].T, y1T, degp,
                   b1.reshape(8, 1), W2.T)
    y2 = y2T8.T

    acc2 = _msg_call_8(y2, src, dst, zeros8).reshape(NC, NPAD, 8)[:, :N, :2]
    outT = _dense3(acc2[0].T, acc2[1].T, y2T8[0:2], degp, b2.reshape(2, 1))
    return outT.T
